# baseline (speedup 1.0000x reference)
import jax
import jax.numpy as jnp
from jax.experimental import pallas as pl
from jax.experimental.pallas import tpu as pltpu

_ROW_BLOCK = 64


def _margin_kernel(t_ref, x_ref, o_ref):
    x = x_ref[...]
    t = t_ref[...]
    col = jax.lax.broadcasted_iota(jnp.int32, x.shape, 1)
    eq = col == t
    neg = jnp.float32(-jnp.inf)
    masked_max = jnp.max(jnp.where(eq, neg, x), axis=1, keepdims=True)
    true_val = jnp.max(jnp.where(eq, x, neg), axis=1, keepdims=True)
    o_ref[...] = masked_max - true_val


def kernel(logits, target):
    B, V = logits.shape
    R = _ROW_BLOCK
    t2 = target.astype(jnp.int32).reshape(B, 1)
    out = pl.pallas_call(
        _margin_kernel,
        grid=(B // R,),
        in_specs=[
            pl.BlockSpec((R, 1), lambda i: (i, 0)),
            pl.BlockSpec((R, V), lambda i: (i, 0)),
        ],
        out_specs=pl.BlockSpec((R, 1), lambda i: (i, 0)),
        out_shape=jax.ShapeDtypeStruct((B, 1), jnp.float32),
        compiler_params=pltpu.CompilerParams(
            dimension_semantics=("parallel",),
        ),
    )(t2, logits)
    return out.reshape(B)


# speedup vs baseline: 1.0039x; 1.0039x over previous
import functools

import jax
import jax.numpy as jnp
from jax.experimental import pallas as pl
from jax.experimental.pallas import tpu as pltpu

_R = 16
_NBUF = 4


def _body(t_ref, x_hbm, o_ref, buf, sems, *, R, NBUF, B, V):
    total = B // R

    def copy(i, slot):
        return pltpu.make_async_copy(
            x_hbm.at[pl.ds(i * R, R), :], buf.at[slot], sems.at[slot]
        )

    for k in range(NBUF - 1):
        copy(k, k).start()

    def step(i, carry):
        slot = jax.lax.rem(i, NBUF)
        copy(i, slot).wait()
        nxt = i + NBUF - 1

        @pl.when(nxt < total)
        def _():
            copy(nxt, jax.lax.rem(nxt, NBUF)).start()

        x = buf[slot]
        t = t_ref[pl.ds(i * R, R), :]
        col = jax.lax.broadcasted_iota(jnp.int32, x.shape, 1)
        eq = col == t
        neg = jnp.float32(-jnp.inf)
        mx = jnp.max(jnp.where(eq, neg, x), axis=1, keepdims=True)
        tv = jnp.max(jnp.where(eq, x, neg), axis=1, keepdims=True)
        o_ref[pl.ds(i * R, R), :] = mx - tv
        return carry

    jax.lax.fori_loop(0, total, step, 0)


def kernel(logits, target):
    B, V = logits.shape
    R, NBUF = _R, _NBUF
    t2 = target.astype(jnp.int32).reshape(B, 1)
    out = pl.pallas_call(
        functools.partial(_body, R=R, NBUF=NBUF, B=B, V=V),
        in_specs=[
            pl.BlockSpec(memory_space=pltpu.VMEM),
            pl.BlockSpec(memory_space=pl.ANY),
        ],
        out_specs=pl.BlockSpec(memory_space=pltpu.VMEM),
        out_shape=jax.ShapeDtypeStruct((B, 1), jnp.float32),
        scratch_shapes=[
            pltpu.VMEM((NBUF, R, V), jnp.float32),
            pltpu.SemaphoreType.DMA((NBUF,)),
        ],
    )(t2, logits)
    return out.reshape(B)
